# baseline (speedup 1.0000x reference)
import functools
import math

import jax
import jax.numpy as jnp
from jax import lax
from jax.experimental import pallas as pl
from jax.experimental.pallas import tpu as pltpu
from jax.experimental.pallas import tpu_sc as plsc

VOCAB = 100000
EMB = 64
HID = 128
BATCH = 1024

TV = 1024
NT = math.ceil(VOCAB / TV)

_NC, _NS = 2, 16
_NW = _NC * _NS
_BPW = BATCH // _NW


@functools.cache
def _get_sc_gather():
    mesh = plsc.VectorSubcoreMesh(core_axis_name="c", subcore_axis_name="s")

    @functools.partial(
        pl.kernel,
        mesh=mesh,
        out_type=jax.ShapeDtypeStruct((BATCH, EMB), jnp.float32),
        scratch_types=[
            pltpu.VMEM((_BPW,), jnp.int32),
            pltpu.VMEM((_BPW, EMB), jnp.float32),
            pltpu.SemaphoreType.DMA,
        ],
        compiler_params=pltpu.CompilerParams(use_tc_tiling_on_sc=False),
    )
    def sc_gather(table_hbm, idx_hbm, out_hbm, idx_v, rows_v, sem):
        wid = lax.axis_index("s") * _NC + lax.axis_index("c")
        base = wid * _BPW
        pltpu.sync_copy(idx_hbm.at[pl.ds(base, _BPW)], idx_v)
        pltpu.async_copy(table_hbm.at[idx_v], rows_v, sem).wait()
        pltpu.sync_copy(rows_v, out_hbm.at[pl.ds(base, _BPW)])

    return sc_gather


_LOG2E = 1.4426950408889634
_LN2 = 0.6931471805599453
_NEG = -1e30


def _stats_body(emb_ref, w1_ref, b1_ref, w2t_ref, b2_ref,
                ht_ref, lse_ref, ht2_ref, m_ref, s_ref, lg_ref):
    t = pl.program_id(0)

    @pl.when(t == 0)
    def _init():
        h = jnp.dot(emb_ref[...], w1_ref[...],
                    preferred_element_type=jnp.float32) + b1_ref[...]
        ht = jnp.maximum(h, 0.0).T
        ht_ref[...] = ht
        ht2_ref[...] = ht * _LOG2E
        m_ref[...] = jnp.full((1, BATCH), _NEG, jnp.float32)
        s_ref[...] = jnp.zeros((1, BATCH), jnp.float32)
        lg_ref[1] = jnp.full((TV, BATCH), _NEG, jnp.float32)

    def fold(lg2):
        m_old = m_ref[...]
        m_new = jnp.maximum(m_old, jnp.max(lg2, axis=0, keepdims=True))
        s_ref[...] = (s_ref[...] * jnp.exp2(m_old - m_new)
                      + jnp.sum(jnp.exp2(lg2 - m_new), axis=0,
                                keepdims=True))
        m_ref[...] = m_new

    fold(lg_ref[(t - 1) % 2])
    lg_ref[t % 2] = jnp.dot(w2t_ref[...].astype(jnp.bfloat16),
                            ht2_ref[...].astype(jnp.bfloat16),
                            preferred_element_type=jnp.float32) \
        + b2_ref[...].T * _LOG2E

    @pl.when(t == NT - 1)
    def _fin():
        row = t * TV + lax.broadcasted_iota(jnp.int32, (TV, 1), 0)
        fold(jnp.where(row < VOCAB, lg_ref[t % 2], _NEG))
        lse_ref[...] = m_ref[...] * _LN2 + jnp.log(s_ref[...])


def _write_body(ht_ref, lse_ref, w2t_ref, b2_ref, out_ref):
    logits = jnp.dot(w2t_ref[...].astype(jnp.bfloat16),
                     ht_ref[...].astype(jnp.bfloat16),
                     preferred_element_type=jnp.float32) + b2_ref[...].T
    out_ref[...] = logits - lse_ref[...]


def kernel(inputs, emb_table, W1, b1, W2, b2):
    b1r = b1.reshape(1, HID)
    b2r = b2.reshape(1, VOCAB)
    W2T = W2.T

    embeds = _get_sc_gather()(emb_table, inputs)

    ht, lse = pl.pallas_call(
        _stats_body,
        grid=(NT,),
        in_specs=[
            pl.BlockSpec((BATCH, EMB), lambda t: (0, 0)),
            pl.BlockSpec((EMB, HID), lambda t: (0, 0)),
            pl.BlockSpec((1, HID), lambda t: (0, 0)),
            pl.BlockSpec((TV, HID), lambda t: (t, 0)),
            pl.BlockSpec((1, TV), lambda t: (0, t)),
        ],
        out_specs=[
            pl.BlockSpec((HID, BATCH), lambda t: (0, 0)),
            pl.BlockSpec((1, BATCH), lambda t: (0, 0)),
        ],
        out_shape=[
            jax.ShapeDtypeStruct((HID, BATCH), jnp.float32),
            jax.ShapeDtypeStruct((1, BATCH), jnp.float32),
        ],
        scratch_shapes=[
            pltpu.VMEM((HID, BATCH), jnp.float32),
            pltpu.VMEM((1, BATCH), jnp.float32),
            pltpu.VMEM((1, BATCH), jnp.float32),
            pltpu.VMEM((2, TV, BATCH), jnp.float32),
        ],
    )(embeds, W1, b1r, W2T, b2r)

    log_probs_t = pl.pallas_call(
        _write_body,
        grid=(NT,),
        in_specs=[
            pl.BlockSpec((HID, BATCH), lambda t: (0, 0)),
            pl.BlockSpec((1, BATCH), lambda t: (0, 0)),
            pl.BlockSpec((TV, HID), lambda t: (t, 0)),
            pl.BlockSpec((1, TV), lambda t: (0, t)),
        ],
        out_specs=pl.BlockSpec((TV, BATCH), lambda t: (t, 0)),
        out_shape=jax.ShapeDtypeStruct((VOCAB, BATCH), jnp.float32),
        compiler_params=pltpu.CompilerParams(
            dimension_semantics=("arbitrary",),
        ),
    )(ht, lse, W2T, b2r)

    return log_probs_t.T


# speedup vs baseline: 1.0819x; 1.0819x over previous
import functools
import math

import jax
import jax.numpy as jnp
from jax import lax
from jax.experimental import pallas as pl
from jax.experimental.pallas import tpu as pltpu
from jax.experimental.pallas import tpu_sc as plsc

VOCAB = 100000
EMB = 64
HID = 128
BATCH = 1024

TV = 1024
NT = math.ceil(VOCAB / TV)

_NC, _NS = 2, 16
_NW = _NC * _NS
_BPW = BATCH // _NW


@functools.cache
def _get_sc_gather():
    mesh = plsc.VectorSubcoreMesh(core_axis_name="c", subcore_axis_name="s")

    @functools.partial(
        pl.kernel,
        mesh=mesh,
        out_type=jax.ShapeDtypeStruct((BATCH, EMB), jnp.float32),
        scratch_types=[
            pltpu.VMEM((_BPW,), jnp.int32),
            pltpu.VMEM((_BPW, EMB), jnp.float32),
            pltpu.SemaphoreType.DMA,
        ],
        compiler_params=pltpu.CompilerParams(use_tc_tiling_on_sc=False),
    )
    def sc_gather(table_hbm, idx_hbm, out_hbm, idx_v, rows_v, sem):
        wid = lax.axis_index("s") * _NC + lax.axis_index("c")
        base = wid * _BPW
        pltpu.sync_copy(idx_hbm.at[pl.ds(base, _BPW)], idx_v)
        pltpu.async_copy(table_hbm.at[idx_v], rows_v, sem).wait()
        pltpu.sync_copy(rows_v, out_hbm.at[pl.ds(base, _BPW)])

    return sc_gather


_LOG2E = 1.4426950408889634
_LN2 = 0.6931471805599453
_NEG = -1e30


def _stats_body(emb_ref, w1_ref, b1_ref, w2t_ref, b2_ref,
                ht_ref, lse_ref, ht2_ref, m_ref, s_ref):
    t = pl.program_id(0)

    @pl.when(t == 0)
    def _init():
        h = jnp.dot(emb_ref[...], w1_ref[...],
                    preferred_element_type=jnp.float32) + b1_ref[...]
        ht = jnp.maximum(h, 0.0).T
        ht_ref[...] = ht
        ht2_ref[...] = ht * _LOG2E
        m_ref[...] = jnp.full((1, BATCH), _NEG, jnp.float32)
        s_ref[...] = jnp.zeros((1, BATCH), jnp.float32)

    lg2 = jnp.dot(w2t_ref[...].astype(jnp.bfloat16),
                  ht2_ref[...].astype(jnp.bfloat16),
                  preferred_element_type=jnp.float32) \
        + b2_ref[...].T * _LOG2E
    row = t * TV + lax.broadcasted_iota(jnp.int32, (TV, 1), 0)
    lg2 = jnp.where(row < VOCAB, lg2, _NEG)
    m_old = m_ref[...]
    m_new = jnp.maximum(m_old, jnp.max(lg2, axis=0, keepdims=True))
    s_ref[...] = (s_ref[...] * jnp.exp2(m_old - m_new)
                  + jnp.sum(jnp.exp2(lg2 - m_new), axis=0, keepdims=True))
    m_ref[...] = m_new

    @pl.when(t == NT - 1)
    def _fin():
        lse_ref[...] = m_ref[...] * _LN2 + jnp.log(s_ref[...])


def _write_body(ht_ref, lse_ref, w2t_ref, b2_ref, out_ref):
    logits = jnp.dot(w2t_ref[...].astype(jnp.bfloat16),
                     ht_ref[...].astype(jnp.bfloat16),
                     preferred_element_type=jnp.float32) + b2_ref[...].T
    out_ref[...] = logits - lse_ref[...]


def kernel(inputs, emb_table, W1, b1, W2, b2):
    b1r = b1.reshape(1, HID)
    b2r = b2.reshape(1, VOCAB)
    W2T = W2.T

    embeds = _get_sc_gather()(emb_table, inputs)

    ht, lse = pl.pallas_call(
        _stats_body,
        grid=(NT,),
        in_specs=[
            pl.BlockSpec((BATCH, EMB), lambda t: (0, 0)),
            pl.BlockSpec((EMB, HID), lambda t: (0, 0)),
            pl.BlockSpec((1, HID), lambda t: (0, 0)),
            pl.BlockSpec((TV, HID), lambda t: (t, 0)),
            pl.BlockSpec((1, TV), lambda t: (0, t)),
        ],
        out_specs=[
            pl.BlockSpec((HID, BATCH), lambda t: (0, 0)),
            pl.BlockSpec((1, BATCH), lambda t: (0, 0)),
        ],
        out_shape=[
            jax.ShapeDtypeStruct((HID, BATCH), jnp.float32),
            jax.ShapeDtypeStruct((1, BATCH), jnp.float32),
        ],
        scratch_shapes=[
            pltpu.VMEM((HID, BATCH), jnp.float32),
            pltpu.VMEM((1, BATCH), jnp.float32),
            pltpu.VMEM((1, BATCH), jnp.float32),
        ],
    )(embeds, W1, b1r, W2T, b2r)

    log_probs_t = pl.pallas_call(
        _write_body,
        grid=(NT,),
        in_specs=[
            pl.BlockSpec((HID, BATCH), lambda t: (0, 0)),
            pl.BlockSpec((1, BATCH), lambda t: (0, 0)),
            pl.BlockSpec((TV, HID), lambda t: (t, 0)),
            pl.BlockSpec((1, TV), lambda t: (0, t)),
        ],
        out_specs=pl.BlockSpec((TV, BATCH), lambda t: (t, 0)),
        out_shape=jax.ShapeDtypeStruct((VOCAB, BATCH), jnp.float32),
        compiler_params=pltpu.CompilerParams(
            dimension_semantics=("arbitrary",),
        ),
    )(ht, lse, W2T, b2r)

    return log_probs_t.T


# speedup vs baseline: 1.1491x; 1.0621x over previous
import functools
import math

import jax
import jax.numpy as jnp
from jax import lax
from jax.experimental import pallas as pl
from jax.experimental.pallas import tpu as pltpu
from jax.experimental.pallas import tpu_sc as plsc

VOCAB = 100000
EMB = 64
HID = 128
BATCH = 1024

TV = 1024
NT = math.ceil(VOCAB / TV)

_NC, _NS = 2, 16
_NW = _NC * _NS
_BPW = BATCH // _NW


_FPW = EMB // _NW
_NCK = BATCH // 128


@functools.cache
def _get_sc_gather():
    mesh = plsc.VectorSubcoreMesh(core_axis_name="c", subcore_axis_name="s")

    @functools.partial(
        pl.kernel,
        mesh=mesh,
        out_type=jax.ShapeDtypeStruct((EMB, BATCH), jnp.float32),
        scratch_types=[
            pltpu.VMEM((BATCH,), jnp.int32),
            pltpu.VMEM((_FPW * _NCK, 128), jnp.int32),
            pltpu.VMEM((_FPW, BATCH), jnp.float32),
            pltpu.SemaphoreType.DMA,
        ],
        compiler_params=pltpu.CompilerParams(use_tc_tiling_on_sc=False),
    )
    def sc_gather(flat_hbm, idx_hbm, out_hbm, iv, ixs, gat, sem):
        wid = lax.axis_index("s") * _NC + lax.axis_index("c")
        f0 = wid * _FPW
        pltpu.sync_copy(idx_hbm, iv)
        for f in range(_FPW):
            base = (f0 + f) * VOCAB
            for j in range(_NCK):
                for k in range(8):
                    ixs[f * _NCK + j, pl.ds(k * 16, 16)] = (
                        iv[pl.ds(j * 128 + k * 16, 16)] + base)
        cps = [
            pltpu.async_copy(flat_hbm.at[ixs.at[f * _NCK + j]],
                             gat.at[f, pl.ds(j * 128, 128)], sem)
            for f in range(_FPW) for j in range(_NCK)
        ]
        for cp in cps:
            cp.wait()
        pltpu.sync_copy(gat, out_hbm.at[pl.ds(f0, _FPW)])

    return sc_gather


_LOG2E = 1.4426950408889634
_LN2 = 0.6931471805599453
_NEG = -1e30


def _stats_body(emb_ref, w1_ref, b1_ref, w2t_ref, b2_ref,
                ht_ref, lse_ref, ht2_ref, m_ref, s_ref):
    t = pl.program_id(0)

    @pl.when(t == 0)
    def _init():
        ht = jnp.maximum(
            jnp.dot(w1_ref[...], emb_ref[...],
                    preferred_element_type=jnp.float32) + b1_ref[...], 0.0)
        ht_ref[...] = ht
        ht2_ref[...] = ht * _LOG2E
        m_ref[...] = jnp.full((1, BATCH), _NEG, jnp.float32)
        s_ref[...] = jnp.zeros((1, BATCH), jnp.float32)

    lg2 = jnp.dot(w2t_ref[...].astype(jnp.bfloat16),
                  ht2_ref[...].astype(jnp.bfloat16),
                  preferred_element_type=jnp.float32) \
        + b2_ref[...].T * _LOG2E
    row = t * TV + lax.broadcasted_iota(jnp.int32, (TV, 1), 0)
    lg2 = jnp.where(row < VOCAB, lg2, _NEG)
    m_old = m_ref[...]
    m_new = jnp.maximum(m_old, jnp.max(lg2, axis=0, keepdims=True))
    s_ref[...] = (s_ref[...] * jnp.exp2(m_old - m_new)
                  + jnp.sum(jnp.exp2(lg2 - m_new), axis=0, keepdims=True))
    m_ref[...] = m_new

    @pl.when(t == NT - 1)
    def _fin():
        lse_ref[...] = m_ref[...] * _LN2 + jnp.log(s_ref[...])


def _write_body(ht_ref, lse_ref, w2t_ref, b2_ref, out_ref):
    logits = jnp.dot(w2t_ref[...].astype(jnp.bfloat16),
                     ht_ref[...].astype(jnp.bfloat16),
                     preferred_element_type=jnp.float32) + b2_ref[...].T
    out_ref[...] = logits - lse_ref[...]


def kernel(inputs, emb_table, W1, b1, W2, b2):
    b1c = b1.reshape(HID, 1)
    b2r = b2.reshape(1, VOCAB)
    W1T = W1.T
    W2T = W2.T
    emb_flat = emb_table.T.reshape(-1)

    embt = _get_sc_gather()(emb_flat, inputs)

    ht, lse = pl.pallas_call(
        _stats_body,
        grid=(NT,),
        in_specs=[
            pl.BlockSpec((EMB, BATCH), lambda t: (0, 0)),
            pl.BlockSpec((HID, EMB), lambda t: (0, 0)),
            pl.BlockSpec((HID, 1), lambda t: (0, 0)),
            pl.BlockSpec((TV, HID), lambda t: (t, 0)),
            pl.BlockSpec((1, TV), lambda t: (0, t)),
        ],
        out_specs=[
            pl.BlockSpec((HID, BATCH), lambda t: (0, 0)),
            pl.BlockSpec((1, BATCH), lambda t: (0, 0)),
        ],
        out_shape=[
            jax.ShapeDtypeStruct((HID, BATCH), jnp.float32),
            jax.ShapeDtypeStruct((1, BATCH), jnp.float32),
        ],
        scratch_shapes=[
            pltpu.VMEM((HID, BATCH), jnp.float32),
            pltpu.VMEM((1, BATCH), jnp.float32),
            pltpu.VMEM((1, BATCH), jnp.float32),
        ],
    )(embt, W1T, b1c, W2T, b2r)

    log_probs_t = pl.pallas_call(
        _write_body,
        grid=(NT,),
        in_specs=[
            pl.BlockSpec((HID, BATCH), lambda t: (0, 0)),
            pl.BlockSpec((1, BATCH), lambda t: (0, 0)),
            pl.BlockSpec((TV, HID), lambda t: (t, 0)),
            pl.BlockSpec((1, TV), lambda t: (0, t)),
        ],
        out_specs=pl.BlockSpec((TV, BATCH), lambda t: (t, 0)),
        out_shape=jax.ShapeDtypeStruct((VOCAB, BATCH), jnp.float32),
        compiler_params=pltpu.CompilerParams(
            dimension_semantics=("arbitrary",),
        ),
    )(ht, lse, W2T, b2r)

    return log_probs_t.T


# speedup vs baseline: 1.1558x; 1.0059x over previous
import functools
import math

import jax
import jax.numpy as jnp
from jax import lax
from jax.experimental import pallas as pl
from jax.experimental.pallas import tpu as pltpu
from jax.experimental.pallas import tpu_sc as plsc

VOCAB = 100000
EMB = 64
HID = 128
BATCH = 1024

TV = 1024
NT = math.ceil(VOCAB / TV)

_NC, _NS = 2, 16
_NW = _NC * _NS
_BPW = BATCH // _NW


_FPW = EMB // _NW
_NCK = BATCH // 128


@functools.cache
def _get_sc_gather():
    mesh = plsc.VectorSubcoreMesh(core_axis_name="c", subcore_axis_name="s")

    @functools.partial(
        pl.kernel,
        mesh=mesh,
        out_type=jax.ShapeDtypeStruct((EMB, BATCH), jnp.float32),
        scratch_types=[
            pltpu.VMEM((BATCH,), jnp.int32),
            pltpu.VMEM((_FPW * _NCK, 128), jnp.int32),
            pltpu.VMEM((_FPW, BATCH), jnp.float32),
            pltpu.SemaphoreType.DMA,
        ],
        compiler_params=pltpu.CompilerParams(use_tc_tiling_on_sc=False),
    )
    def sc_gather(flat_hbm, idx_hbm, out_hbm, iv, ixs, gat, sem):
        wid = lax.axis_index("s") * _NC + lax.axis_index("c")
        f0 = wid * _FPW
        pltpu.sync_copy(idx_hbm, iv)
        for f in range(_FPW):
            base = (f0 + f) * VOCAB
            for j in range(_NCK):
                for k in range(8):
                    ixs[f * _NCK + j, pl.ds(k * 16, 16)] = (
                        iv[pl.ds(j * 128 + k * 16, 16)] + base)
        cps = [
            pltpu.async_copy(flat_hbm.at[ixs.at[f * _NCK + j]],
                             gat.at[f, pl.ds(j * 128, 128)], sem)
            for f in range(_FPW) for j in range(_NCK)
        ]
        for cp in cps:
            cp.wait()
        pltpu.sync_copy(gat, out_hbm.at[pl.ds(f0, _FPW)])

    return sc_gather


_LOG2E = 1.4426950408889634
_LN2 = 0.6931471805599453
_NEG = -1e30


def _stats_body(emb_ref, w1_ref, b1_ref, w2t_ref, b2_ref,
                ht_ref, lse_ref, ht2_ref, m_ref, s_ref):
    t = pl.program_id(0)

    @pl.when(t == 0)
    def _init():
        ht = jnp.maximum(
            jnp.dot(w1_ref[...], emb_ref[...],
                    preferred_element_type=jnp.float32) + b1_ref[...], 0.0)
        ht_ref[...] = ht
        ht2_ref[...] = ht * _LOG2E
        m_ref[...] = jnp.full((1, BATCH), _NEG, jnp.float32)
        s_ref[...] = jnp.zeros((1, BATCH), jnp.float32)

    def fold(lg2):
        m_old = m_ref[...]
        m_new = jnp.maximum(m_old, jnp.max(lg2, axis=0, keepdims=True))
        s_ref[...] = (s_ref[...] * jnp.exp2(m_old - m_new)
                      + jnp.sum(jnp.exp2(lg2 - m_new), axis=0,
                                keepdims=True))
        m_ref[...] = m_new

    ht2 = ht2_ref[...].astype(jnp.bfloat16)
    b2t = b2_ref[...].T * _LOG2E
    row = t * TV + lax.broadcasted_iota(jnp.int32, (TV, 1), 0)
    HTV = TV // 2
    lgs = [
        jnp.dot(w2t_ref[pl.ds(i * HTV, HTV), :].astype(jnp.bfloat16), ht2,
                preferred_element_type=jnp.float32)
        + b2t[i * HTV:(i + 1) * HTV, :]
        for i in range(2)
    ]
    for i in range(2):
        fold(jnp.where(row[i * HTV:(i + 1) * HTV, :] < VOCAB, lgs[i], _NEG))

    @pl.when(t == NT - 1)
    def _fin():
        lse_ref[...] = m_ref[...] * _LN2 + jnp.log(s_ref[...])


def _write_body(ht_ref, lse_ref, w2t_ref, b2_ref, out_ref):
    logits = jnp.dot(w2t_ref[...].astype(jnp.bfloat16),
                     ht_ref[...].astype(jnp.bfloat16),
                     preferred_element_type=jnp.float32) + b2_ref[...].T
    out_ref[...] = logits - lse_ref[...]


def kernel(inputs, emb_table, W1, b1, W2, b2):
    b1c = b1.reshape(HID, 1)
    b2r = b2.reshape(1, VOCAB)
    W1T = W1.T
    W2T = W2.T
    emb_flat = emb_table.T.reshape(-1)

    embt = _get_sc_gather()(emb_flat, inputs)

    ht, lse = pl.pallas_call(
        _stats_body,
        grid=(NT,),
        in_specs=[
            pl.BlockSpec((EMB, BATCH), lambda t: (0, 0)),
            pl.BlockSpec((HID, EMB), lambda t: (0, 0)),
            pl.BlockSpec((HID, 1), lambda t: (0, 0)),
            pl.BlockSpec((TV, HID), lambda t: (t, 0)),
            pl.BlockSpec((1, TV), lambda t: (0, t)),
        ],
        out_specs=[
            pl.BlockSpec((HID, BATCH), lambda t: (0, 0)),
            pl.BlockSpec((1, BATCH), lambda t: (0, 0)),
        ],
        out_shape=[
            jax.ShapeDtypeStruct((HID, BATCH), jnp.float32),
            jax.ShapeDtypeStruct((1, BATCH), jnp.float32),
        ],
        scratch_shapes=[
            pltpu.VMEM((HID, BATCH), jnp.float32),
            pltpu.VMEM((1, BATCH), jnp.float32),
            pltpu.VMEM((1, BATCH), jnp.float32),
        ],
    )(embt, W1T, b1c, W2T, b2r)

    log_probs_t = pl.pallas_call(
        _write_body,
        grid=(NT,),
        in_specs=[
            pl.BlockSpec((HID, BATCH), lambda t: (0, 0)),
            pl.BlockSpec((1, BATCH), lambda t: (0, 0)),
            pl.BlockSpec((TV, HID), lambda t: (t, 0)),
            pl.BlockSpec((1, TV), lambda t: (0, t)),
        ],
        out_specs=pl.BlockSpec((TV, BATCH), lambda t: (t, 0)),
        out_shape=jax.ShapeDtypeStruct((VOCAB, BATCH), jnp.float32),
        compiler_params=pltpu.CompilerParams(
            dimension_semantics=("arbitrary",),
        ),
    )(ht, lse, W2T, b2r)

    return log_probs_t.T


# speedup vs baseline: 1.1968x; 1.0355x over previous
import functools
import math

import jax
import jax.numpy as jnp
from jax import lax
from jax.experimental import pallas as pl
from jax.experimental.pallas import tpu as pltpu
from jax.experimental.pallas import tpu_sc as plsc

VOCAB = 100000
EMB = 64
HID = 128
BATCH = 1024

TV = 1024
NT = math.ceil(VOCAB / TV)

_NC, _NS = 2, 16
_NW = _NC * _NS
_BPW = BATCH // _NW


_FPW = EMB // _NW
_NCK = BATCH // 128


@functools.cache
def _get_sc_gather():
    mesh = plsc.VectorSubcoreMesh(core_axis_name="c", subcore_axis_name="s")

    @functools.partial(
        pl.kernel,
        mesh=mesh,
        out_type=jax.ShapeDtypeStruct((EMB, BATCH), jnp.float32),
        scratch_types=[
            pltpu.VMEM((BATCH,), jnp.int32),
            pltpu.VMEM((_FPW * _NCK, 128), jnp.int32),
            pltpu.VMEM((_FPW, BATCH), jnp.float32),
            pltpu.SemaphoreType.DMA,
        ],
        compiler_params=pltpu.CompilerParams(use_tc_tiling_on_sc=False),
    )
    def sc_gather(flat_hbm, idx_hbm, out_hbm, iv, ixs, gat, sem):
        wid = lax.axis_index("s") * _NC + lax.axis_index("c")
        f0 = wid * _FPW
        pltpu.sync_copy(idx_hbm, iv)
        for f in range(_FPW):
            base = (f0 + f) * VOCAB
            for j in range(_NCK):
                for k in range(8):
                    ixs[f * _NCK + j, pl.ds(k * 16, 16)] = (
                        iv[pl.ds(j * 128 + k * 16, 16)] + base)
        cps = [
            pltpu.async_copy(flat_hbm.at[ixs.at[f * _NCK + j]],
                             gat.at[f, pl.ds(j * 128, 128)], sem)
            for f in range(_FPW) for j in range(_NCK)
        ]
        for cp in cps:
            cp.wait()
        pltpu.sync_copy(gat, out_hbm.at[pl.ds(f0, _FPW)])

    return sc_gather


_LOG2E = 1.4426950408889634
_LN2 = 0.6931471805599453
_NEG = -1e30


def _stats_body(emb_ref, w1_ref, b1_ref, w2t_ref, b2_ref,
                ht_ref, lse_ref, ht2_ref, m_ref, s_ref):
    t = pl.program_id(0)

    @pl.when(t == 0)
    def _init():
        ht = jnp.maximum(
            jnp.dot(w1_ref[...], emb_ref[...],
                    preferred_element_type=jnp.float32) + b1_ref[...], 0.0)
        ht_ref[...] = ht
        ht2_ref[...] = ht * _LOG2E
        m_ref[...] = jnp.full((1, BATCH), _NEG, jnp.float32)
        s_ref[...] = jnp.zeros((1, BATCH), jnp.float32)

    def fold(lg2):
        m_old = m_ref[...]
        m_new = jnp.maximum(m_old, jnp.max(lg2, axis=0, keepdims=True))
        s_ref[...] = (s_ref[...] * jnp.exp2(m_old - m_new)
                      + jnp.sum(jnp.exp2(lg2 - m_new), axis=0,
                                keepdims=True))
        m_ref[...] = m_new

    HTV = TV // 2

    def half_logits(i):
        ht2 = ht2_ref[...].astype(jnp.bfloat16)
        b2t = b2_ref[...].T * _LOG2E
        return jnp.dot(w2t_ref[pl.ds(i * HTV, HTV), :].astype(jnp.bfloat16),
                       ht2, preferred_element_type=jnp.float32) \
            + b2t[i * HTV:(i + 1) * HTV, :]

    @pl.when(t < NT - 1)
    def _hot():
        lgs = [half_logits(0), half_logits(1)]
        fold(lgs[0])
        fold(lgs[1])

    @pl.when(t == NT - 1)
    def _last():
        row = t * TV + lax.broadcasted_iota(jnp.int32, (TV, 1), 0)
        lgs = [half_logits(0), half_logits(1)]
        for i in range(2):
            fold(jnp.where(row[i * HTV:(i + 1) * HTV, :] < VOCAB,
                           lgs[i], _NEG))
        lse_ref[...] = m_ref[...] * _LN2 + jnp.log(s_ref[...])


def _write_body(ht_ref, lse_ref, w2t_ref, b2_ref, out_ref):
    logits = jnp.dot(w2t_ref[...].astype(jnp.bfloat16),
                     ht_ref[...].astype(jnp.bfloat16),
                     preferred_element_type=jnp.float32) + b2_ref[...].T
    out_ref[...] = logits - lse_ref[...]


def kernel(inputs, emb_table, W1, b1, W2, b2):
    b1c = b1.reshape(HID, 1)
    b2r = b2.reshape(1, VOCAB)
    W1T = W1.T
    W2T = W2.T
    emb_flat = emb_table.T.reshape(-1)

    embt = _get_sc_gather()(emb_flat, inputs)

    ht, lse = pl.pallas_call(
        _stats_body,
        grid=(NT,),
        in_specs=[
            pl.BlockSpec((EMB, BATCH), lambda t: (0, 0)),
            pl.BlockSpec((HID, EMB), lambda t: (0, 0)),
            pl.BlockSpec((HID, 1), lambda t: (0, 0)),
            pl.BlockSpec((TV, HID), lambda t: (t, 0)),
            pl.BlockSpec((1, TV), lambda t: (0, t)),
        ],
        out_specs=[
            pl.BlockSpec((HID, BATCH), lambda t: (0, 0)),
            pl.BlockSpec((1, BATCH), lambda t: (0, 0)),
        ],
        out_shape=[
            jax.ShapeDtypeStruct((HID, BATCH), jnp.float32),
            jax.ShapeDtypeStruct((1, BATCH), jnp.float32),
        ],
        scratch_shapes=[
            pltpu.VMEM((HID, BATCH), jnp.float32),
            pltpu.VMEM((1, BATCH), jnp.float32),
            pltpu.VMEM((1, BATCH), jnp.float32),
        ],
    )(embt, W1T, b1c, W2T, b2r)

    log_probs_t = pl.pallas_call(
        _write_body,
        grid=(NT,),
        in_specs=[
            pl.BlockSpec((HID, BATCH), lambda t: (0, 0)),
            pl.BlockSpec((1, BATCH), lambda t: (0, 0)),
            pl.BlockSpec((TV, HID), lambda t: (t, 0)),
            pl.BlockSpec((1, TV), lambda t: (0, t)),
        ],
        out_specs=pl.BlockSpec((TV, BATCH), lambda t: (t, 0)),
        out_shape=jax.ShapeDtypeStruct((VOCAB, BATCH), jnp.float32),
        compiler_params=pltpu.CompilerParams(
            dimension_semantics=("arbitrary",),
        ),
    )(ht, lse, W2T, b2r)

    return log_probs_t.T
